# baseline (speedup 1.0000x reference)
import functools

import jax
import jax.numpy as jnp
from jax.experimental import pallas as pl
from jax.experimental.pallas import tpu as pltpu

TOPK = 8
EPS = 1e-5
_NEG = -3.0e38


def _fused_kernel(x_ref, w1_ref, b1_ref, gamma_ref, beta_ref, w2_ref, b2_ref,
                  out_ref, h_ref, w1b_ref, stats_ref, *, nb, bb1, bb2, inv_b):
    i = pl.program_id(0)

    @pl.when(i == 0)
    def _():
        w1b_ref[...] = w1_ref[...].astype(jnp.bfloat16)

    @pl.when(i < nb)
    def _phase1():
        nsub = 1
        sub = bb1 // nsub
        for c in range(nsub):
            h = jnp.dot(x_ref[pl.ds(c * sub, sub), :].astype(jnp.bfloat16),
                        w1b_ref[...], preferred_element_type=jnp.float32)
            h = jnp.maximum(h + b1_ref[...], 0.0)
            h_ref[pl.ds(i * bb1 + c * sub, sub), :] = h
            s = jnp.sum(h, axis=0, keepdims=True)
            ss = jnp.sum(h * h, axis=0, keepdims=True)

            @pl.when(jnp.logical_and(i == 0, c == 0))
            def _():
                stats_ref[0:1, :] = s
                stats_ref[1:2, :] = ss

            @pl.when(jnp.logical_or(i > 0, c > 0))
            def _():
                stats_ref[0:1, :] = stats_ref[0:1, :] + s
                stats_ref[1:2, :] = stats_ref[1:2, :] + ss

    @pl.when(i >= nb)
    def _phase2():
        j = i - nb
        mean = stats_ref[0:1, :] * inv_b
        var = jnp.maximum(stats_ref[1:2, :] * inv_b - mean * mean, 0.0)
        scale = gamma_ref[...] * jax.lax.rsqrt(var + EPS)
        shift = beta_ref[...] - mean * scale
        h = h_ref[pl.ds(j * bb2, bb2), :]
        hn = h * scale + shift
        w2b = w2_ref[...].astype(jnp.bfloat16)
        logits_rm = jnp.dot(hn.astype(jnp.bfloat16), w2b,
                            preferred_element_type=jnp.float32) + b2_ref[...]
        logits = logits_rm.T

        nexp = logits.shape[0]
        iota = jax.lax.broadcasted_iota(
            jnp.int32, logits.shape, 0).astype(jnp.float32)
        work = logits
        sel = jnp.zeros(logits.shape, jnp.bool_)
        rowmax = None
        for _ in range(TOPK):
            m = jnp.max(work, axis=0, keepdims=True)
            if rowmax is None:
                rowmax = m
            cand = work >= m
            first = jnp.min(jnp.where(cand, iota, float(nexp)),
                            axis=0, keepdims=True)
            pick = iota == first
            sel = jnp.logical_or(sel, pick)
            work = jnp.where(pick, _NEG, work)

        p = jnp.where(sel, jnp.exp(logits - rowmax), 0.0)
        out_ref[...] = (p / jnp.sum(p, axis=0, keepdims=True)).T


def kernel(x, W1, b1, gamma, beta, W2, b2):
    B, D = x.shape
    H = W1.shape[1]
    E = W2.shape[1]
    BB1 = 256
    BB2 = min(2048, B)
    nb = B // BB1
    nb2 = B // BB2

    b1r = b1.reshape(1, H)
    gammar = gamma.reshape(1, H)
    betar = beta.reshape(1, H)
    b2r = b2.reshape(1, E)

    out = pl.pallas_call(
        functools.partial(_fused_kernel, nb=nb, bb1=BB1, bb2=BB2,
                          inv_b=1.0 / B),
        grid=(nb + nb2,),
        in_specs=[
            pl.BlockSpec((BB1, D), lambda i: (jnp.minimum(i, nb - 1), 0)),
            pl.BlockSpec((D, H), lambda i: (0, 0)),
            pl.BlockSpec((1, H), lambda i: (0, 0)),
            pl.BlockSpec((1, H), lambda i: (0, 0)),
            pl.BlockSpec((1, H), lambda i: (0, 0)),
            pl.BlockSpec((H, E), lambda i: (0, 0)),
            pl.BlockSpec((1, E), lambda i: (0, 0)),
        ],
        out_specs=pl.BlockSpec((BB2, E), lambda i: (jnp.maximum(i - nb, 0), 0)),
        out_shape=jax.ShapeDtypeStruct((B, E), jnp.float32),
        scratch_shapes=[
            pltpu.VMEM((B, H), jnp.float32),
            pltpu.VMEM((D, H), jnp.bfloat16),
            pltpu.VMEM((8, H), jnp.float32),
        ],
        compiler_params=pltpu.CompilerParams(
            dimension_semantics=("arbitrary",)),
    )(x, W1, b1r, gammar, betar, W2, b2r)
    return out


# speedup vs baseline: 1.1190x; 1.1190x over previous
import functools

import jax
import jax.numpy as jnp
from jax.experimental import pallas as pl
from jax.experimental.pallas import tpu as pltpu

TOPK = 8
EPS = 1e-5
_NEG = -3.0e38


def _fused_kernel(x_ref, w1_ref, b1_ref, gamma_ref, beta_ref, w2_ref, b2_ref,
                  out_ref, h_ref, w1b_ref, stats_ref, *, nb, bb1, bb2, inv_b):
    i = pl.program_id(0)

    @pl.when(i == 0)
    def _():
        w1b_ref[...] = w1_ref[...].astype(jnp.bfloat16)

    @pl.when(i < nb)
    def _phase1():
        nsub = 1
        sub = bb1 // nsub
        for c in range(nsub):
            h = jnp.dot(x_ref[pl.ds(c * sub, sub), :].astype(jnp.bfloat16),
                        w1b_ref[...], preferred_element_type=jnp.float32)
            h = jnp.maximum(h + b1_ref[...], 0.0)
            h_ref[pl.ds(i * bb1 + c * sub, sub), :] = h
            s = jnp.sum(h, axis=0, keepdims=True)
            ss = jnp.sum(h * h, axis=0, keepdims=True)

            @pl.when(jnp.logical_and(i == 0, c == 0))
            def _():
                stats_ref[0:1, :] = s
                stats_ref[1:2, :] = ss

            @pl.when(jnp.logical_or(i > 0, c > 0))
            def _():
                stats_ref[0:1, :] = stats_ref[0:1, :] + s
                stats_ref[1:2, :] = stats_ref[1:2, :] + ss

    @pl.when(i >= nb)
    def _phase2():
        j = i - nb
        mean = stats_ref[0:1, :] * inv_b
        var = jnp.maximum(stats_ref[1:2, :] * inv_b - mean * mean, 0.0)
        scale = gamma_ref[...] * jax.lax.rsqrt(var + EPS)
        shift = beta_ref[...] - mean * scale
        h = h_ref[pl.ds(j * bb2, bb2), :]
        hn = h * scale + shift
        w2b = w2_ref[...].astype(jnp.bfloat16)
        logits_rm = jnp.dot(hn.astype(jnp.bfloat16), w2b,
                            preferred_element_type=jnp.float32) + b2_ref[...]
        logits = logits_rm.T

        nexp = logits.shape[0]
        iota = jax.lax.broadcasted_iota(
            jnp.int32, logits.shape, 0).astype(jnp.float32)
        work = logits
        sel = jnp.zeros(logits.shape, jnp.bool_)
        rowmax = None
        for _ in range(TOPK):
            m = jnp.max(work, axis=0, keepdims=True)
            if rowmax is None:
                rowmax = m
            cand = work >= m
            first = jnp.min(jnp.where(cand, iota, float(nexp)),
                            axis=0, keepdims=True)
            pick = iota == first
            sel = jnp.logical_or(sel, pick)
            work = jnp.where(pick, _NEG, work)

        p = jnp.where(sel, jnp.exp(logits - rowmax), 0.0)
        out_ref[...] = (p / jnp.sum(p, axis=0, keepdims=True)).T


def kernel(x, W1, b1, gamma, beta, W2, b2):
    B, D = x.shape
    H = W1.shape[1]
    E = W2.shape[1]
    BB1 = 512
    BB2 = min(4096, B)
    nb = B // BB1
    nb2 = B // BB2

    b1r = b1.reshape(1, H)
    gammar = gamma.reshape(1, H)
    betar = beta.reshape(1, H)
    b2r = b2.reshape(1, E)

    out = pl.pallas_call(
        functools.partial(_fused_kernel, nb=nb, bb1=BB1, bb2=BB2,
                          inv_b=1.0 / B),
        grid=(nb + nb2,),
        in_specs=[
            pl.BlockSpec((BB1, D), lambda i: (jnp.minimum(i, nb - 1), 0)),
            pl.BlockSpec((D, H), lambda i: (0, 0)),
            pl.BlockSpec((1, H), lambda i: (0, 0)),
            pl.BlockSpec((1, H), lambda i: (0, 0)),
            pl.BlockSpec((1, H), lambda i: (0, 0)),
            pl.BlockSpec((H, E), lambda i: (0, 0)),
            pl.BlockSpec((1, E), lambda i: (0, 0)),
        ],
        out_specs=pl.BlockSpec((BB2, E), lambda i: (jnp.maximum(i - nb, 0), 0)),
        out_shape=jax.ShapeDtypeStruct((B, E), jnp.float32),
        scratch_shapes=[
            pltpu.VMEM((B, H), jnp.float32),
            pltpu.VMEM((D, H), jnp.bfloat16),
            pltpu.VMEM((8, H), jnp.float32),
        ],
        compiler_params=pltpu.CompilerParams(
            dimension_semantics=("arbitrary",)),
    )(x, W1, b1r, gammar, betar, W2, b2r)
    return out
